# baseline (speedup 1.0000x reference)
import functools

import jax
import jax.numpy as jnp
from jax import lax
from jax.experimental import pallas as pl
from jax.experimental.pallas import tpu as pltpu
from jax.experimental.pallas import tpu_sc as plsc

B = 16384
D = 32
NC = 2
NS = 16
NW = NC * NS
BPW = B // NW
CHUNK = 128
NCHUNK = BPW // CHUNK

TCHUNK = 65536


def _matvec_body(w_ref, x_ref, o_ref):
    s = jax.lax.dot_general(
        w_ref[...], x_ref[...], (((1,), (0,)), ((), ())),
        preferred_element_type=jnp.float32)
    o_ref[...] = s.reshape(1, 1, TCHUNK)


def _scores(xT, w):
    n = xT.shape[1]
    grid = (n + TCHUNK - 1) // TCHUNK
    out = pl.pallas_call(
        _matvec_body,
        grid=(grid,),
        in_specs=[
            pl.BlockSpec((1, D), lambda n: (0, 0)),
            pl.BlockSpec((D, TCHUNK), lambda n: (0, n)),
        ],
        out_specs=pl.BlockSpec((1, 1, TCHUNK), lambda n: (n, 0, 0)),
        out_shape=jax.ShapeDtypeStruct((grid, 1, TCHUNK), jnp.float32),
    )(w.reshape(1, D), xT)
    return out.reshape(grid * TCHUNK // 128, 128)


def _gather_body(idx_h, s_h, add_h, out_h,
                 raw, q, buf0, buf1, av, outv, sem, *maybe_shared):
    wid = lax.axis_index("s") * NC + lax.axis_index("c")
    base = wid * BPW
    per_elem = add_h.shape[0] == B

    pltpu.sync_copy(idx_h.at[pl.ds(base, BPW)], raw)
    if per_elem:
        pltpu.sync_copy(add_h.at[pl.ds(base, BPW)], av)
    else:
        pltpu.sync_copy(add_h, av)
    iota = lax.iota(jnp.int32, 16)

    if maybe_shared:
        s_shared = maybe_shared[0]
        rows = s_h.shape[0] // NS
        sid = lax.axis_index("s")
        pltpu.sync_copy(s_h.at[pl.ds(sid * rows, rows)],
                        s_shared.at[pl.ds(sid * rows, rows)])
        src = s_shared
    else:
        src = s_h

    def shift(g, carry):
        q[pl.ds(g * 16, 16)] = lax.shift_right_logical(
            raw[pl.ds(g * 16, 16)], 7)
        return carry

    lax.fori_loop(0, BPW // 16, shift, 0)
    if maybe_shared:
        plsc.subcore_barrier()

    bufs = [buf0, buf1]

    def fire(c):
        sl = pl.ds(c * CHUNK, CHUNK)
        return pltpu.async_copy(src.at[q.at[sl]], bufs[c % 2], sem)

    cp = fire(0)
    for c in range(NCHUNK):
        nxt = fire(c + 1) if c + 1 < NCHUNK else None
        cp.wait()
        buf = bufs[c % 2]

        def group(j, carry):
            row = iota + j * 16
            lane = raw[pl.ds(c * CHUNK + j * 16, 16)] & 127
            val = plsc.load_gather(buf, [row, lane])
            if per_elem:
                add = av[pl.ds(c * CHUNK + j * 16, 16)]
            else:
                add = av[pl.ds(0, 16)][0]
            outv[pl.ds(c * CHUNK + j * 16, 16)] = val + add
            return carry

        lax.fori_loop(0, CHUNK // 16, group, 0)
        cp = nxt

    pltpu.sync_copy(outv, out_h.at[pl.ds(base, BPW)])


def _gather_add(idx, scores, addend, stage_shared=False):
    scratch = [
        pltpu.VMEM((BPW,), jnp.int32),
        pltpu.VMEM((BPW,), jnp.int32),
        pltpu.VMEM((CHUNK, 128), jnp.float32),
        pltpu.VMEM((CHUNK, 128), jnp.float32),
        pltpu.VMEM((BPW if addend.shape[0] == B else 16,), jnp.float32),
        pltpu.VMEM((BPW,), jnp.float32),
        pltpu.SemaphoreType.DMA,
    ]
    if stage_shared:
        scratch.append(pltpu.VMEM_SHARED(scores.shape, jnp.float32))
    run = pl.kernel(
        _gather_body,
        mesh=plsc.VectorSubcoreMesh(core_axis_name="c", subcore_axis_name="s"),
        compiler_params=pltpu.CompilerParams(needs_layout_passes=False),
        out_type=jax.ShapeDtypeStruct((B,), jnp.float32),
        scratch_types=scratch,
    )
    return run(idx, scores, addend)


@functools.partial(jax.jit, static_argnames=())
def kernel(users, movies, user_table, movie_table, W, b):
    users = users.astype(jnp.int32)
    movies = movies.astype(jnp.int32)
    sm = _scores(movie_table.T, W[0, D:])
    su = _scores(user_table.T, W[0, :D])
    bias = jnp.broadcast_to(b, (16,)).astype(jnp.float32)
    partial = _gather_add(movies, sm, bias, stage_shared=True)
    out = _gather_add(users, su, partial, stage_shared=True)
    return out.reshape(B, 1)


# speedup vs baseline: 1.0524x; 1.0524x over previous
import functools

import jax
import jax.numpy as jnp
from jax import lax
from jax.experimental import pallas as pl
from jax.experimental.pallas import tpu as pltpu
from jax.experimental.pallas import tpu_sc as plsc

B = 16384
D = 32
NC = 2
NS = 16
NW = NC * NS
BPW = B // NW
CHUNK = 128
NCHUNK = BPW // CHUNK

TCHUNK = 65536


def _matvec_body(w_ref, x_ref, o_ref):
    s = jax.lax.dot_general(
        w_ref[...], x_ref[...], (((1,), (0,)), ((), ())),
        preferred_element_type=jnp.float32)
    o_ref[...] = s.reshape(1, 1, TCHUNK)


def _scores(xT, w):
    n = xT.shape[1]
    grid = (n + TCHUNK - 1) // TCHUNK
    out = pl.pallas_call(
        _matvec_body,
        grid=(grid,),
        in_specs=[
            pl.BlockSpec((1, D), lambda n: (0, 0)),
            pl.BlockSpec((D, TCHUNK), lambda n: (0, n)),
        ],
        out_specs=pl.BlockSpec((1, 1, TCHUNK), lambda n: (n, 0, 0)),
        out_shape=jax.ShapeDtypeStruct((grid, 1, TCHUNK), jnp.float32),
    )(w.reshape(1, D), xT)
    return out.reshape(grid * TCHUNK // 128, 128)


def _gather_body(idx_h, s_h, add_h, out_h,
                 raw, q, buf0, buf1, av, outv, sem, *maybe_shared):
    wid = lax.axis_index("s") * NC + lax.axis_index("c")
    base = wid * BPW
    per_elem = add_h.shape[0] == B

    pltpu.sync_copy(idx_h.at[pl.ds(base, BPW)], raw)
    if per_elem:
        pltpu.sync_copy(add_h.at[pl.ds(base, BPW)], av)
    else:
        pltpu.sync_copy(add_h, av)
    iota = lax.iota(jnp.int32, 16)

    if maybe_shared:
        s_shared = maybe_shared[0]
        rows = s_h.shape[0] // NS
        sid = lax.axis_index("s")
        pltpu.sync_copy(s_h.at[pl.ds(sid * rows, rows)],
                        s_shared.at[pl.ds(sid * rows, rows)])
        src = s_shared
    else:
        src = s_h

    def shift(g, carry):
        q[pl.ds(g * 16, 16)] = lax.shift_right_logical(
            raw[pl.ds(g * 16, 16)], 7)
        return carry

    lax.fori_loop(0, BPW // 16, shift, 0)
    if maybe_shared:
        plsc.subcore_barrier()

    bufs = [buf0, buf1]

    def fire(c):
        sl = pl.ds(c * CHUNK, CHUNK)
        return pltpu.async_copy(src.at[q.at[sl]], bufs[c % 2], sem)

    cp = fire(0)
    for c in range(NCHUNK):
        nxt = fire(c + 1) if c + 1 < NCHUNK else None
        cp.wait()
        buf = bufs[c % 2]

        def group(j, carry):
            row = iota + j * 16
            lane = raw[pl.ds(c * CHUNK + j * 16, 16)] & 127
            val = plsc.load_gather(buf, [row, lane])
            if per_elem:
                add = av[pl.ds(c * CHUNK + j * 16, 16)]
            else:
                add = av[pl.ds(0, 16)][0]
            outv[pl.ds(c * CHUNK + j * 16, 16)] = val + add
            return carry

        lax.fori_loop(0, CHUNK // 16, group, 0)
        cp = nxt

    pltpu.sync_copy(outv, out_h.at[pl.ds(base, BPW)])


def _gather_add(idx, scores, addend, stage_shared=False):
    scratch = [
        pltpu.VMEM((BPW,), jnp.int32),
        pltpu.VMEM((BPW,), jnp.int32),
        pltpu.VMEM((CHUNK, 128), jnp.float32),
        pltpu.VMEM((CHUNK, 128), jnp.float32),
        pltpu.VMEM((BPW if addend.shape[0] == B else 16,), jnp.float32),
        pltpu.VMEM((BPW,), jnp.float32),
        pltpu.SemaphoreType.DMA,
    ]
    if stage_shared:
        scratch.append(pltpu.VMEM_SHARED(scores.shape, jnp.float32))
    run = pl.kernel(
        _gather_body,
        mesh=plsc.VectorSubcoreMesh(core_axis_name="c", subcore_axis_name="s"),
        compiler_params=pltpu.CompilerParams(needs_layout_passes=False),
        out_type=jax.ShapeDtypeStruct((B,), jnp.float32),
        scratch_types=scratch,
    )
    return run(idx, scores, addend)


@functools.partial(jax.jit, static_argnames=())
def kernel(users, movies, user_table, movie_table, W, b):
    users = users.astype(jnp.int32)
    movies = movies.astype(jnp.int32)
    sm = _scores(movie_table.T, W[0, D:])
    su = _scores(user_table.T, W[0, :D])
    bias = jnp.broadcast_to(b, (16,)).astype(jnp.float32)
    partial = _gather_add(movies, sm, bias, stage_shared=True)
    out = _gather_add(users, su, partial)
    return out.reshape(B, 1)
